# baseline (speedup 1.0000x reference)
import functools

import jax
import jax.numpy as jnp
from jax import lax
from jax.experimental import pallas as pl
from jax.experimental.pallas import tpu as pltpu
from jax.experimental.pallas import tpu_sc as plsc

E = 8
K = 2
BK = 256
HB = 512


def _router_body(x_ref, wr_ref, tp_ref, ti_ref):
    logits = jnp.dot(x_ref[...].astype(jnp.bfloat16),
                     wr_ref[...].astype(jnp.bfloat16),
                     preferred_element_type=jnp.float32)
    col = lax.broadcasted_iota(jnp.int32, logits.shape, 1)
    logits = jnp.where(col < E, logits, jnp.float32(-1e30))
    m = jnp.max(logits, axis=1, keepdims=True)
    ex = jnp.exp(logits - m)
    probs = ex / jnp.sum(ex, axis=1, keepdims=True)
    p1 = jnp.max(probs, axis=1, keepdims=True)
    i1 = jnp.min(jnp.where(probs == p1, col, E), axis=1, keepdims=True)
    probs2 = jnp.where(col == i1, jnp.float32(-1.0), probs)
    p2 = jnp.max(probs2, axis=1, keepdims=True)
    i2 = jnp.min(jnp.where(probs2 == p2, col, E), axis=1, keepdims=True)
    tp_ref[...] = jnp.where(col == 0, p1, jnp.where(col == 1, p2, 0.0))
    ti_ref[...] = jnp.where(col == 0, i1, jnp.where(col == 1, i2, 0))


def _router(xf, wr_pad):
    T = xf.shape[0]
    return pl.pallas_call(
        _router_body,
        out_shape=[jax.ShapeDtypeStruct((T, 128), jnp.float32),
                   jax.ShapeDtypeStruct((T, 128), jnp.int32)],
    )(xf, wr_pad)


def _make_dispatch(T, P, D):
    NW = 32
    tpw = T // NW
    mesh = plsc.VectorSubcoreMesh(core_axis_name="c", subcore_axis_name="s")

    @functools.partial(
        pl.kernel, mesh=mesh,
        out_type=jax.ShapeDtypeStruct((P, D), jnp.float32),
        scratch_types=[
            pltpu.VMEM((tpw,), jnp.int32),
            pltpu.VMEM((tpw,), jnp.int32),
            pltpu.VMEM((tpw, D), jnp.float32),
            pltpu.SemaphoreType.DMA,
        ],
    )
    def dk(x_hbm, pe_hbm, po_hbm, out_hbm, pe_v, po_v, rows_v, sem):
        wid = lax.axis_index("s") * 2 + lax.axis_index("c")
        base = wid * tpw
        pltpu.sync_copy(pe_hbm.at[pl.ds(base, tpw)], pe_v)
        pltpu.sync_copy(po_hbm.at[pl.ds(base, tpw)], po_v)
        pltpu.sync_copy(x_hbm.at[pl.ds(base, tpw)], rows_v)
        c1 = pltpu.async_copy(rows_v, out_hbm.at[pe_v], sem)
        c2 = pltpu.async_copy(rows_v, out_hbm.at[po_v], sem)
        c1.wait()
        c2.wait()

    return dk


def _make_gather(N, D, C):
    NW = 32
    rows_per = N // NW
    nch = rows_per // C
    assert rows_per % C == 0 and N % NW == 0 and C % 8 == 0
    mesh = plsc.VectorSubcoreMesh(core_axis_name="c", subcore_axis_name="s")

    @functools.partial(
        pl.kernel, mesh=mesh,
        out_type=jax.ShapeDtypeStruct((N, D), jnp.float32),
        scratch_types=[
            pltpu.VMEM((C,), jnp.int32),
            pltpu.VMEM((C, D), jnp.float32),
            pltpu.SemaphoreType.DMA,
        ],
    )
    def gk(table_hbm, idx_hbm, out_hbm, idx_v, rows_v, sem):
        wid = lax.axis_index("s") * 2 + lax.axis_index("c")
        base0 = wid * rows_per

        def body(c, carry):
            base = base0 + c * C
            pltpu.sync_copy(idx_hbm.at[pl.ds(base, C)], idx_v)
            pltpu.async_copy(table_hbm.at[idx_v], rows_v, sem).wait()
            pltpu.sync_copy(rows_v, out_hbm.at[pl.ds(base, C)])
            return carry

        lax.fori_loop(0, nch, body, 0)

    return gk


def _mlp_body(be_ref, xg_ref, win_ref, bin_ref, wout_ref, bout_ref,
              out_ref, *, NB):
    @pl.when(pl.program_id(0) < be_ref[NB])
    def _():
        xb = xg_ref[...].astype(jnp.bfloat16)
        a = jnp.dot(xb, win_ref[0], preferred_element_type=jnp.float32)
        a = a + bin_ref[0]
        hb = jax.nn.gelu(a).astype(jnp.bfloat16)
        out_ref[...] = (jnp.dot(hb, wout_ref[0],
                                preferred_element_type=jnp.float32)
                        + bout_ref[0])


def _grouped_mlp(be, gathered, w_in, b_in, w_out, b_out, NB):
    P, D = gathered.shape
    H = w_in.shape[2]
    grid_spec = pltpu.PrefetchScalarGridSpec(
        num_scalar_prefetch=1,
        grid=(NB,),
        in_specs=[
            pl.BlockSpec((BK, D), lambda b, be: (b, 0)),
            pl.BlockSpec((1, D, H), lambda b, be: (be[b], 0, 0)),
            pl.BlockSpec((1, 1, H), lambda b, be: (be[b], 0, 0)),
            pl.BlockSpec((1, H, D), lambda b, be: (be[b], 0, 0)),
            pl.BlockSpec((1, 1, D), lambda b, be: (be[b], 0, 0)),
        ],
        out_specs=pl.BlockSpec((BK, D), lambda b, be: (b, 0)),
    )
    return pl.pallas_call(
        functools.partial(_mlp_body, NB=NB),
        grid_spec=grid_spec,
        out_shape=jax.ShapeDtypeStruct((P, D), jnp.float32),
    )(be, gathered, w_in, b_in.reshape(E, 1, H), w_out,
      b_out.reshape(E, 1, D))


def _sum_body(a_ref, b_ref, ga_ref, gb_ref, o_ref):
    o_ref[...] = a_ref[...] * ga_ref[...] + b_ref[...] * gb_ref[...]


def _pair_sum(ffu, g0, g1, T, D):
    bt = 512
    nblk = T // bt
    return pl.pallas_call(
        _sum_body,
        grid=(nblk,),
        in_specs=[
            pl.BlockSpec((bt, D), lambda i: (i, 0)),
            pl.BlockSpec((bt, D), lambda i: (i + nblk, 0)),
            pl.BlockSpec((bt, 1), lambda i: (i, 0)),
            pl.BlockSpec((bt, 1), lambda i: (i, 0)),
        ],
        out_specs=pl.BlockSpec((bt, D), lambda i: (i, 0)),
        out_shape=jax.ShapeDtypeStruct((T, D), jnp.float32),
    )(ffu, ffu, g0, g1)


def kernel(x, Wr, W_in, b_in, W_out, b_out):
    B, S, D = x.shape
    T = B * S
    TK = T * K
    NB = TK // BK + E - 1
    P = NB * BK
    xf = x.reshape(T, D)

    wr_pad = jnp.zeros((D, 128), Wr.dtype).at[:, :E].set(Wr)
    tp_pad, ti_pad = _router(xf, wr_pad)
    ti = ti_pad[:, :K].reshape(TK)

    oh = (ti[:, None] == jnp.arange(E, dtype=jnp.int32)[None, :]).astype(jnp.int32)
    rank = jnp.sum((jnp.cumsum(oh, axis=0) - oh) * oh, axis=1)
    counts = jnp.sum(oh, axis=0)
    nb = (counts + BK - 1) // BK
    cnb = jnp.cumsum(nb)
    pad_off = BK * (cnb - nb)
    pos = (jnp.sum(oh * pad_off[None, :], axis=1) + rank).astype(jnp.int32)

    b_ar = jnp.arange(NB, dtype=jnp.int32)
    be = jnp.sum((b_ar[:, None] >= cnb[None, :]).astype(jnp.int32), axis=1)
    last_e = jnp.max(jnp.where(nb > 0, jnp.arange(E, dtype=jnp.int32), 0))
    be = jnp.where(b_ar < cnb[-1], be, last_e).astype(jnp.int32)
    be = jnp.concatenate([be, cnb[-1:].astype(jnp.int32)])

    pos2 = pos.reshape(T, K)
    gathered = _make_dispatch(T, P, D)(xf, pos2[:, 0], pos2[:, 1])

    w_in_bf = W_in.astype(jnp.bfloat16)
    w_out_bf = W_out.astype(jnp.bfloat16)
    ff = _grouped_mlp(be, gathered, w_in_bf, b_in, w_out_bf, b_out, NB)

    posperm = pos2.T.reshape(TK)
    ffu = _make_gather(TK, D, 32)(ff, posperm)
    out = _pair_sum(ffu, tp_pad[:, 0:1], tp_pad[:, 1:2], T, D)
    return out.reshape(B, S, D)


# speedup vs baseline: 1.0545x; 1.0545x over previous
import functools

import jax
import jax.numpy as jnp
from jax import lax
from jax.experimental import pallas as pl
from jax.experimental.pallas import tpu as pltpu
from jax.experimental.pallas import tpu_sc as plsc

E = 8
K = 2
BK = 512
HB = 512


def _router_body(x_ref, wr_ref, tp_ref, pe_ref, po_ref, cnb_ref):
    T = x_ref.shape[0]
    logits = jnp.dot(x_ref[...].astype(jnp.bfloat16),
                     wr_ref[...].astype(jnp.bfloat16),
                     preferred_element_type=jnp.float32)
    col = lax.broadcasted_iota(jnp.int32, logits.shape, 1)
    logits = jnp.where(col < E, logits, jnp.float32(-1e30))
    m = jnp.max(logits, axis=1, keepdims=True)
    ex = jnp.exp(logits - m)
    probs = ex / jnp.sum(ex, axis=1, keepdims=True)
    p1 = jnp.max(probs, axis=1, keepdims=True)
    i1 = jnp.min(jnp.where(probs == p1, col, E), axis=1, keepdims=True)
    probs2 = jnp.where(col == i1, jnp.float32(-1.0), probs)
    p2 = jnp.max(probs2, axis=1, keepdims=True)
    i2 = jnp.min(jnp.where(probs2 == p2, col, E), axis=1, keepdims=True)
    tp_ref[...] = jnp.where(col == 0, p1, jnp.where(col == 1, p2, 0.0))

    cnt = ((col == i1) | (col == i2)).astype(jnp.int32)
    csum = cnt
    sh = 1
    while sh < T:
        top = jax.lax.slice(csum, (0, 0), (T - sh, 128))
        z = jnp.zeros((sh, 128), jnp.int32)
        csum = csum + jnp.concatenate([z, top], axis=0)
        sh *= 2
    excl = csum - cnt
    counts = jax.lax.slice(csum, (T - 1, 0), (T, 128))
    nb = jax.lax.shift_right_logical(counts + (BK - 1), 9)
    lcol = lax.broadcasted_iota(jnp.int32, (128, 128), 0)
    mcol = lax.broadcasted_iota(jnp.int32, (128, 128), 1)
    tri = ((lcol <= mcol) & (lcol < E)).astype(jnp.float32)
    cnb = jnp.dot(nb.astype(jnp.float32), tri,
                  preferred_element_type=jnp.float32)
    pad_off = ((cnb.astype(jnp.int32) - nb) * BK)
    tgt = pad_off + excl
    pe_ref[...] = jnp.sum(jnp.where(col == i1, tgt, 0), axis=1)
    po_ref[...] = jnp.sum(jnp.where(col == i2, tgt, 0), axis=1)
    cnb_ref[...] = cnb.astype(jnp.int32)


def _router(xf, wr_pad):
    T = xf.shape[0]
    return pl.pallas_call(
        _router_body,
        out_shape=[jax.ShapeDtypeStruct((T, 128), jnp.float32),
                   jax.ShapeDtypeStruct((T,), jnp.int32),
                   jax.ShapeDtypeStruct((T,), jnp.int32),
                   jax.ShapeDtypeStruct((1, 128), jnp.int32)],
    )(xf, wr_pad)


def _make_dispatch(T, P, D):
    NW = 32
    tpw = T // NW
    mesh = plsc.VectorSubcoreMesh(core_axis_name="c", subcore_axis_name="s")

    @functools.partial(
        pl.kernel, mesh=mesh,
        out_type=jax.ShapeDtypeStruct((P, D), jnp.float32),
        scratch_types=[
            pltpu.VMEM((tpw,), jnp.int32),
            pltpu.VMEM((tpw,), jnp.int32),
            pltpu.VMEM((tpw, D), jnp.float32),
            pltpu.SemaphoreType.DMA,
        ],
    )
    def dk(x_hbm, pe_hbm, po_hbm, out_hbm, pe_v, po_v, rows_v, sem):
        wid = lax.axis_index("s") * 2 + lax.axis_index("c")
        base = wid * tpw
        pltpu.sync_copy(pe_hbm.at[pl.ds(base, tpw)], pe_v)
        pltpu.sync_copy(po_hbm.at[pl.ds(base, tpw)], po_v)
        pltpu.sync_copy(x_hbm.at[pl.ds(base, tpw)], rows_v)
        c1 = pltpu.async_copy(rows_v, out_hbm.at[pe_v], sem)
        c2 = pltpu.async_copy(rows_v, out_hbm.at[po_v], sem)
        c1.wait()
        c2.wait()

    return dk


def _make_gather(N, D, C):
    NW = 32
    rows_per = N // NW
    nch = rows_per // C
    assert rows_per % C == 0 and N % NW == 0 and C % 8 == 0
    mesh = plsc.VectorSubcoreMesh(core_axis_name="c", subcore_axis_name="s")

    @functools.partial(
        pl.kernel, mesh=mesh,
        out_type=jax.ShapeDtypeStruct((N, D), jnp.float32),
        scratch_types=[
            pltpu.VMEM((C,), jnp.int32),
            pltpu.VMEM((C, D), jnp.float32),
            pltpu.SemaphoreType.DMA,
        ],
    )
    def gk(table_hbm, idx_hbm, out_hbm, idx_v, rows_v, sem):
        wid = lax.axis_index("s") * 2 + lax.axis_index("c")
        base0 = wid * rows_per

        def body(c, carry):
            base = base0 + c * C
            pltpu.sync_copy(idx_hbm.at[pl.ds(base, C)], idx_v)
            pltpu.async_copy(table_hbm.at[idx_v], rows_v, sem).wait()
            pltpu.sync_copy(rows_v, out_hbm.at[pl.ds(base, C)])
            return carry

        lax.fori_loop(0, nch, body, 0)

    return gk


def _mlp_body(be_ref, xg_ref, win_ref, bin_ref, wout_ref, bout_ref,
              out_ref, *, NB):
    @pl.when(pl.program_id(0) < be_ref[NB])
    def _():
        xb = xg_ref[...].astype(jnp.bfloat16)
        a = jnp.dot(xb, win_ref[0], preferred_element_type=jnp.float32)
        a = a + bin_ref[0]
        hb = jax.nn.gelu(a).astype(jnp.bfloat16)
        out_ref[...] = (jnp.dot(hb, wout_ref[0],
                                preferred_element_type=jnp.float32)
                        + bout_ref[0])


def _grouped_mlp(be, gathered, w_in, b_in, w_out, b_out, NB):
    P, D = gathered.shape
    H = w_in.shape[2]
    grid_spec = pltpu.PrefetchScalarGridSpec(
        num_scalar_prefetch=1,
        grid=(NB,),
        in_specs=[
            pl.BlockSpec((BK, D), lambda b, be: (b, 0)),
            pl.BlockSpec((1, D, H), lambda b, be: (be[b], 0, 0)),
            pl.BlockSpec((1, 1, H), lambda b, be: (be[b], 0, 0)),
            pl.BlockSpec((1, H, D), lambda b, be: (be[b], 0, 0)),
            pl.BlockSpec((1, 1, D), lambda b, be: (be[b], 0, 0)),
        ],
        out_specs=pl.BlockSpec((BK, D), lambda b, be: (b, 0)),
    )
    return pl.pallas_call(
        functools.partial(_mlp_body, NB=NB),
        grid_spec=grid_spec,
        out_shape=jax.ShapeDtypeStruct((P, D), jnp.float32),
    )(be, gathered, w_in, b_in.reshape(E, 1, H), w_out,
      b_out.reshape(E, 1, D))


def _sum_body(a_ref, b_ref, ga_ref, gb_ref, o_ref):
    o_ref[...] = a_ref[...] * ga_ref[...] + b_ref[...] * gb_ref[...]


def _pair_sum(ffu, g0, g1, T, D):
    bt = 512
    nblk = T // bt
    return pl.pallas_call(
        _sum_body,
        grid=(nblk,),
        in_specs=[
            pl.BlockSpec((bt, D), lambda i: (i, 0)),
            pl.BlockSpec((bt, D), lambda i: (i + nblk, 0)),
            pl.BlockSpec((bt, 1), lambda i: (i, 0)),
            pl.BlockSpec((bt, 1), lambda i: (i, 0)),
        ],
        out_specs=pl.BlockSpec((bt, D), lambda i: (i, 0)),
        out_shape=jax.ShapeDtypeStruct((T, D), jnp.float32),
    )(ffu, ffu, g0, g1)


def kernel(x, Wr, W_in, b_in, W_out, b_out):
    B, S, D = x.shape
    T = B * S
    TK = T * K
    NB = TK // BK + E - 1
    P = NB * BK
    xf = x.reshape(T, D)

    wr_pad = jnp.zeros((D, 128), Wr.dtype).at[:, :E].set(Wr)
    tp_pad, pe, po, cnb_row = _router(xf, wr_pad)

    cnb = cnb_row[0, :E]
    nb = cnb - jnp.concatenate([jnp.zeros((1,), jnp.int32), cnb[:-1]])
    b_ar = jnp.arange(NB, dtype=jnp.int32)
    be = jnp.sum((b_ar[:, None] >= cnb[None, :]).astype(jnp.int32), axis=1)
    last_e = jnp.max(jnp.where(nb > 0, jnp.arange(E, dtype=jnp.int32), 0))
    be = jnp.where(b_ar < cnb[-1], be, last_e).astype(jnp.int32)
    be = jnp.concatenate([be, cnb[-1:]])

    gathered = _make_dispatch(T, P, D)(xf, pe, po)

    w_in_bf = W_in.astype(jnp.bfloat16)
    w_out_bf = W_out.astype(jnp.bfloat16)
    ff = _grouped_mlp(be, gathered, w_in_bf, b_in, w_out_bf, b_out, NB)

    posperm = jnp.concatenate([pe, po])
    ffu = _make_gather(TK, D, 32)(ff, posperm)
    out = _pair_sum(ffu, tp_pad[:, 0:1], tp_pad[:, 1:2], T, D)
    return out.reshape(B, S, D)


# speedup vs baseline: 1.3366x; 1.2675x over previous
import functools

import jax
import jax.numpy as jnp
from jax import lax
from jax.experimental import pallas as pl
from jax.experimental.pallas import tpu as pltpu
from jax.experimental.pallas import tpu_sc as plsc

E = 8
K = 2
BK = 256


def _router_body(x_ref, wr_ref, tp_ref, pp_ref, be_ref, *, NB):
    T = x_ref.shape[0]
    logits = jnp.dot(x_ref[...].astype(jnp.bfloat16),
                     wr_ref[...].astype(jnp.bfloat16),
                     preferred_element_type=jnp.float32)
    col = lax.broadcasted_iota(jnp.int32, logits.shape, 1)
    logits = jnp.where(col < E, logits, jnp.float32(-1e30))
    m = jnp.max(logits, axis=1, keepdims=True)
    ex = jnp.exp(logits - m)
    probs = ex / jnp.sum(ex, axis=1, keepdims=True)
    p1 = jnp.max(probs, axis=1, keepdims=True)
    i1 = jnp.min(jnp.where(probs == p1, col, E), axis=1, keepdims=True)
    probs2 = jnp.where(col == i1, jnp.float32(-1.0), probs)
    p2 = jnp.max(probs2, axis=1, keepdims=True)
    i2 = jnp.min(jnp.where(probs2 == p2, col, E), axis=1, keepdims=True)
    tp_ref[...] = jnp.where(col == 0, p1, jnp.where(col == 1, p2, 0.0))

    cnt = ((col == i1) | (col == i2)).astype(jnp.int32)
    csum = cnt
    sh = 1
    while sh < T:
        top = jax.lax.slice(csum, (0, 0), (T - sh, 128))
        z = jnp.zeros((sh, 128), jnp.int32)
        csum = csum + jnp.concatenate([z, top], axis=0)
        sh *= 2
    excl = csum - cnt
    counts = jax.lax.slice(csum, (T - 1, 0), (T, 128))
    nb = (counts + (BK - 1)) // BK
    lcol = lax.broadcasted_iota(jnp.int32, (128, 128), 0)
    mcol = lax.broadcasted_iota(jnp.int32, (128, 128), 1)
    tri = ((lcol <= mcol) & (lcol < E)).astype(jnp.float32)
    cnbf = jnp.dot(nb.astype(jnp.float32), tri,
                   preferred_element_type=jnp.float32)
    cnb = cnbf.astype(jnp.int32)
    pad_off = (cnb - nb) * BK
    tgt = pad_off + excl
    pe = jnp.sum(jnp.where(col == i1, tgt, 0), axis=1)
    po = jnp.sum(jnp.where(col == i2, tgt, 0), axis=1)
    pp_ref[pl.ds(0, T)] = pe
    pp_ref[pl.ds(T, T)] = po

    nused = jnp.max(jnp.where(mcol >= E - 1, cnb[0][None, :], 0))
    rowb = jnp.minimum(lcol, nused - 1)
    bemat = ((mcol < E) & (cnb[0][None, :] <= rowb)).astype(jnp.int32)
    bevec = jnp.sum(bemat, axis=1)
    bevec = jnp.where(lax.iota(jnp.int32, 128) == NB, nused, bevec)
    be_ref[...] = jax.lax.slice(bevec, (0,), (NB + 1,))


def _router(xf, wr_pad, NB):
    T = xf.shape[0]
    return pl.pallas_call(
        functools.partial(_router_body, NB=NB),
        out_shape=[jax.ShapeDtypeStruct((T, 128), jnp.float32),
                   jax.ShapeDtypeStruct((K * T,), jnp.int32),
                   jax.ShapeDtypeStruct((NB + 1,), jnp.int32)],
    )(xf, wr_pad)


def _make_dispatch(T, P, D):
    NW = 32
    tpw = T // NW
    mesh = plsc.VectorSubcoreMesh(core_axis_name="c", subcore_axis_name="s")

    @functools.partial(
        pl.kernel, mesh=mesh,
        out_type=jax.ShapeDtypeStruct((P, D), jnp.float32),
        scratch_types=[
            pltpu.VMEM((tpw,), jnp.int32),
            pltpu.VMEM((tpw,), jnp.int32),
            pltpu.VMEM((tpw, D), jnp.float32),
            pltpu.SemaphoreType.DMA,
        ],
    )
    def dk(x_hbm, pp_hbm, out_hbm, pe_v, po_v, rows_v, sem):
        wid = lax.axis_index("s") * 2 + lax.axis_index("c")
        base = wid * tpw
        pltpu.sync_copy(pp_hbm.at[pl.ds(base, tpw)], pe_v)
        pltpu.sync_copy(pp_hbm.at[pl.ds(T + base, tpw)], po_v)
        pltpu.sync_copy(x_hbm.at[pl.ds(base, tpw)], rows_v)
        c1 = pltpu.async_copy(rows_v, out_hbm.at[pe_v], sem)
        c2 = pltpu.async_copy(rows_v, out_hbm.at[po_v], sem)
        c1.wait()
        c2.wait()

    return dk


def _make_gather(N, D, C):
    NW = 32
    rows_per = N // NW
    nch = rows_per // C
    assert rows_per % C == 0 and N % NW == 0 and C % 8 == 0
    mesh = plsc.VectorSubcoreMesh(core_axis_name="c", subcore_axis_name="s")

    @functools.partial(
        pl.kernel, mesh=mesh,
        out_type=jax.ShapeDtypeStruct((N, D), jnp.float32),
        scratch_types=[
            pltpu.VMEM((C,), jnp.int32),
            pltpu.VMEM((C, D), jnp.float32),
            pltpu.SemaphoreType.DMA,
        ],
    )
    def gk(table_hbm, idx_hbm, out_hbm, idx_v, rows_v, sem):
        wid = lax.axis_index("s") * 2 + lax.axis_index("c")
        base0 = wid * rows_per

        def body(c, carry):
            base = base0 + c * C
            pltpu.sync_copy(idx_hbm.at[pl.ds(base, C)], idx_v)
            pltpu.async_copy(table_hbm.at[idx_v], rows_v, sem).wait()
            pltpu.sync_copy(rows_v, out_hbm.at[pl.ds(base, C)])
            return carry

        lax.fori_loop(0, nch, body, 0)

    return gk


def _mlp_body(be_ref, xg_ref, win_ref, bin_ref, wout_ref, bout_ref,
              out_ref, *, NB):
    @pl.when(pl.program_id(0) < be_ref[NB])
    def _():
        a = jnp.dot(xg_ref[...], win_ref[0], preferred_element_type=jnp.float32)
        a = a + bin_ref[0]
        h = jax.nn.gelu(a)
        out_ref[...] = (jnp.dot(h, wout_ref[0],
                                preferred_element_type=jnp.float32)
                        + bout_ref[0])


def _grouped_mlp(be, gathered, w_in, b_in, w_out, b_out, NB):
    P, D = gathered.shape
    H = w_in.shape[2]
    grid_spec = pltpu.PrefetchScalarGridSpec(
        num_scalar_prefetch=1,
        grid=(NB,),
        in_specs=[
            pl.BlockSpec((BK, D), lambda b, be: (jnp.minimum(b, be[NB] - 1), 0)),
            pl.BlockSpec((1, D, H), lambda b, be: (be[b], 0, 0)),
            pl.BlockSpec((1, 1, H), lambda b, be: (be[b], 0, 0)),
            pl.BlockSpec((1, H, D), lambda b, be: (be[b], 0, 0)),
            pl.BlockSpec((1, 1, D), lambda b, be: (be[b], 0, 0)),
        ],
        out_specs=pl.BlockSpec((BK, D), lambda b, be: (b, 0)),
    )
    return pl.pallas_call(
        functools.partial(_mlp_body, NB=NB),
        grid_spec=grid_spec,
        out_shape=jax.ShapeDtypeStruct((P, D), jnp.float32),
    )(be, gathered, w_in, b_in.reshape(E, 1, H), w_out,
      b_out.reshape(E, 1, D))


def _sum_body(a_ref, b_ref, ga_ref, gb_ref, o_ref):
    o_ref[...] = a_ref[...] * ga_ref[...] + b_ref[...] * gb_ref[...]


def _pair_sum(ffu, g0, g1, T, D):
    bt = 512
    nblk = T // bt
    return pl.pallas_call(
        _sum_body,
        grid=(nblk,),
        in_specs=[
            pl.BlockSpec((bt, D), lambda i: (i, 0)),
            pl.BlockSpec((bt, D), lambda i: (i + nblk, 0)),
            pl.BlockSpec((bt, 1), lambda i: (i, 0)),
            pl.BlockSpec((bt, 1), lambda i: (i, 0)),
        ],
        out_specs=pl.BlockSpec((bt, D), lambda i: (i, 0)),
        out_shape=jax.ShapeDtypeStruct((T, D), jnp.float32),
    )(ffu, ffu, g0, g1)


def kernel(x, Wr, W_in, b_in, W_out, b_out):
    B, S, D = x.shape
    T = B * S
    TK = T * K
    NB = TK // BK + E - 1
    P = NB * BK
    xf = x.reshape(T, D)

    wr_pad = jnp.zeros((D, 128), Wr.dtype).at[:, :E].set(Wr)
    tp_pad, posperm, be = _router(xf, wr_pad, NB)

    gathered = _make_dispatch(T, P, D)(xf, posperm)

    ff = _grouped_mlp(be, gathered, W_in, b_in, W_out, b_out, NB)

    ffu = _make_gather(TK, D, 32)(ff, posperm)
    out = _pair_sum(ffu, tp_pad[:, 0:1], tp_pad[:, 1:2], T, D)
    return out.reshape(B, S, D)


# speedup vs baseline: 1.3521x; 1.0116x over previous
import functools

import jax
import jax.numpy as jnp
from jax import lax
from jax.experimental import pallas as pl
from jax.experimental.pallas import tpu as pltpu
from jax.experimental.pallas import tpu_sc as plsc

E = 8
K = 2
BK = 256


def _router_body(x_ref, wr_ref, tp_ref, pp_ref, be_ref, *, NB):
    T = x_ref.shape[0]
    lg8 = jnp.dot(x_ref[...].astype(jnp.bfloat16),
                  wr_ref[...].astype(jnp.bfloat16),
                  preferred_element_type=jnp.float32)
    col = lax.broadcasted_iota(jnp.int32, (T, 128), 1)
    logits = jnp.where(col < E,
                       lax.pad(lg8, jnp.float32(0.0), ((0, 0, 0), (0, 128 - E, 0))),
                       jnp.float32(-1e30))
    m = jnp.max(logits, axis=1, keepdims=True)
    ex = jnp.exp(logits - m)
    probs = ex / jnp.sum(ex, axis=1, keepdims=True)
    p1 = jnp.max(probs, axis=1, keepdims=True)
    i1 = jnp.min(jnp.where(probs == p1, col, E), axis=1, keepdims=True)
    probs2 = jnp.where(col == i1, jnp.float32(-1.0), probs)
    p2 = jnp.max(probs2, axis=1, keepdims=True)
    i2 = jnp.min(jnp.where(probs2 == p2, col, E), axis=1, keepdims=True)
    tp_ref[...] = jnp.where(col == 0, p1, jnp.where(col == 1, p2, 0.0))

    cnt = ((col == i1) | (col == i2)).astype(jnp.int32)
    csum = cnt
    sh = 1
    while sh < T:
        top = jax.lax.slice(csum, (0, 0), (T - sh, 128))
        z = jnp.zeros((sh, 128), jnp.int32)
        csum = csum + jnp.concatenate([z, top], axis=0)
        sh *= 2
    excl = csum - cnt
    counts = jax.lax.slice(csum, (T - 1, 0), (T, 128))
    nb = (counts + (BK - 1)) // BK
    lcol = lax.broadcasted_iota(jnp.int32, (128, 128), 0)
    mcol = lax.broadcasted_iota(jnp.int32, (128, 128), 1)
    tri = ((lcol <= mcol) & (lcol < E)).astype(jnp.float32)
    cnbf = jnp.dot(nb.astype(jnp.float32), tri,
                   preferred_element_type=jnp.float32)
    cnb = cnbf.astype(jnp.int32)
    pad_off = (cnb - nb) * BK
    tgt = pad_off + excl
    pe = jnp.sum(jnp.where(col == i1, tgt, 0), axis=1)
    po = jnp.sum(jnp.where(col == i2, tgt, 0), axis=1)
    pp_ref[pl.ds(0, T)] = pe
    pp_ref[pl.ds(T, T)] = po

    nused = jnp.max(jnp.where(mcol >= E - 1, cnb[0][None, :], 0))
    rowb = jnp.minimum(lcol, nused - 1)
    bemat = ((mcol < E) & (cnb[0][None, :] <= rowb)).astype(jnp.int32)
    bevec = jnp.sum(bemat, axis=1)
    bevec = jnp.where(lax.iota(jnp.int32, 128) == NB, nused, bevec)
    be_ref[...] = jax.lax.slice(bevec, (0,), (NB + 1,))


def _router(xf, wr_pad, NB):
    T = xf.shape[0]
    return pl.pallas_call(
        functools.partial(_router_body, NB=NB),
        out_shape=[jax.ShapeDtypeStruct((T, 128), jnp.float32),
                   jax.ShapeDtypeStruct((K * T,), jnp.int32),
                   jax.ShapeDtypeStruct((NB + 1,), jnp.int32)],
    )(xf, wr_pad)


def _make_dispatch(T, P, D):
    NW = 32
    tpw = T // NW
    mesh = plsc.VectorSubcoreMesh(core_axis_name="c", subcore_axis_name="s")

    @functools.partial(
        pl.kernel, mesh=mesh,
        out_type=jax.ShapeDtypeStruct((P, D), jnp.float32),
        scratch_types=[
            pltpu.VMEM((tpw,), jnp.int32),
            pltpu.VMEM((tpw,), jnp.int32),
            pltpu.VMEM((tpw, D), jnp.float32),
            pltpu.SemaphoreType.DMA,
        ],
    )
    def dk(x_hbm, pp_hbm, out_hbm, pe_v, po_v, rows_v, sem):
        wid = lax.axis_index("s") * 2 + lax.axis_index("c")
        base = wid * tpw
        pltpu.sync_copy(pp_hbm.at[pl.ds(base, tpw)], pe_v)
        pltpu.sync_copy(pp_hbm.at[pl.ds(T + base, tpw)], po_v)
        pltpu.sync_copy(x_hbm.at[pl.ds(base, tpw)], rows_v)
        c1 = pltpu.async_copy(rows_v, out_hbm.at[pe_v], sem)
        c2 = pltpu.async_copy(rows_v, out_hbm.at[po_v], sem)
        c1.wait()
        c2.wait()

    return dk


def _make_gather(N, D, C):
    NW = 32
    rows_per = N // NW
    nch = rows_per // C
    assert rows_per % C == 0 and N % NW == 0 and C % 8 == 0
    mesh = plsc.VectorSubcoreMesh(core_axis_name="c", subcore_axis_name="s")

    @functools.partial(
        pl.kernel, mesh=mesh,
        out_type=jax.ShapeDtypeStruct((N, D), jnp.float32),
        scratch_types=[
            pltpu.VMEM((C,), jnp.int32),
            pltpu.VMEM((C, D), jnp.float32),
            pltpu.SemaphoreType.DMA,
        ],
    )
    def gk(table_hbm, idx_hbm, out_hbm, idx_v, rows_v, sem):
        wid = lax.axis_index("s") * 2 + lax.axis_index("c")
        base0 = wid * rows_per

        def body(c, carry):
            base = base0 + c * C
            pltpu.sync_copy(idx_hbm.at[pl.ds(base, C)], idx_v)
            pltpu.async_copy(table_hbm.at[idx_v], rows_v, sem).wait()
            pltpu.sync_copy(rows_v, out_hbm.at[pl.ds(base, C)])
            return carry

        lax.fori_loop(0, nch, body, 0)

    return gk


def _mlp_body(be_ref, xg_ref, win_ref, bin_ref, wout_ref, bout_ref,
              out_ref, *, NB):
    @pl.when(pl.program_id(0) < be_ref[NB])
    def _():
        a = jnp.dot(xg_ref[...], win_ref[0], preferred_element_type=jnp.float32)
        a = a + bin_ref[0]
        h = jax.nn.gelu(a)
        out_ref[...] = (jnp.dot(h, wout_ref[0],
                                preferred_element_type=jnp.float32)
                        + bout_ref[0])


def _grouped_mlp(be, gathered, w_in, b_in, w_out, b_out, NB):
    P, D = gathered.shape
    H = w_in.shape[2]
    grid_spec = pltpu.PrefetchScalarGridSpec(
        num_scalar_prefetch=1,
        grid=(NB,),
        in_specs=[
            pl.BlockSpec((BK, D), lambda b, be: (jnp.minimum(b, be[NB] - 1), 0)),
            pl.BlockSpec((1, D, H), lambda b, be: (be[b], 0, 0)),
            pl.BlockSpec((1, 1, H), lambda b, be: (be[b], 0, 0)),
            pl.BlockSpec((1, H, D), lambda b, be: (be[b], 0, 0)),
            pl.BlockSpec((1, 1, D), lambda b, be: (be[b], 0, 0)),
        ],
        out_specs=pl.BlockSpec((BK, D), lambda b, be: (b, 0)),
    )
    return pl.pallas_call(
        functools.partial(_mlp_body, NB=NB),
        grid_spec=grid_spec,
        out_shape=jax.ShapeDtypeStruct((P, D), jnp.float32),
    )(be, gathered, w_in, b_in.reshape(E, 1, H), w_out,
      b_out.reshape(E, 1, D))


def _sum_body(a_ref, b_ref, g_ref, o_ref):
    g = g_ref[...]
    o_ref[...] = (a_ref[...] * jax.lax.slice(g, (0, 0), (g.shape[0], 1))
                  + b_ref[...] * jax.lax.slice(g, (0, 1), (g.shape[0], 2)))


def _pair_sum(ffu, tp, T, D):
    bt = 512
    nblk = T // bt
    return pl.pallas_call(
        _sum_body,
        grid=(nblk,),
        in_specs=[
            pl.BlockSpec((bt, D), lambda i: (i, 0)),
            pl.BlockSpec((bt, D), lambda i: (i + nblk, 0)),
            pl.BlockSpec((bt, 128), lambda i: (i, 0)),
        ],
        out_specs=pl.BlockSpec((bt, D), lambda i: (i, 0)),
        out_shape=jax.ShapeDtypeStruct((T, D), jnp.float32),
    )(ffu, ffu, tp)


def kernel(x, Wr, W_in, b_in, W_out, b_out):
    B, S, D = x.shape
    T = B * S
    TK = T * K
    NB = TK // BK + E - 1
    P = NB * BK
    xf = x.reshape(T, D)

    tp_pad, posperm, be = _router(xf, Wr, NB)

    gathered = _make_dispatch(T, P, D)(xf, posperm)

    ff = _grouped_mlp(be, gathered, W_in, b_in, W_out, b_out, NB)

    ffu = _make_gather(TK, D, 32)(ff, posperm)
    out = _pair_sum(ffu, tp_pad, T, D)
    return out.reshape(B, S, D)


# speedup vs baseline: 1.3761x; 1.0178x over previous
import functools

import jax
import jax.numpy as jnp
from jax import lax
from jax.experimental import pallas as pl
from jax.experimental.pallas import tpu as pltpu
from jax.experimental.pallas import tpu_sc as plsc

E = 8
K = 2
BK = 256


def _router_body(x_ref, wr_ref, tp_ref, pp_ref, be_ref, *, NB):
    T = x_ref.shape[0]
    lg8 = jnp.dot(x_ref[...].astype(jnp.bfloat16),
                  wr_ref[...].astype(jnp.bfloat16),
                  preferred_element_type=jnp.float32)
    col = lax.broadcasted_iota(jnp.int32, (T, 128), 1)
    logits = jnp.where(col < E,
                       lax.pad(lg8, jnp.float32(0.0), ((0, 0, 0), (0, 128 - E, 0))),
                       jnp.float32(-1e30))
    m = jnp.max(logits, axis=1, keepdims=True)
    ex = jnp.exp(logits - m)
    probs = ex / jnp.sum(ex, axis=1, keepdims=True)
    p1 = jnp.max(probs, axis=1, keepdims=True)
    i1 = jnp.min(jnp.where(probs == p1, col, E), axis=1, keepdims=True)
    probs2 = jnp.where(col == i1, jnp.float32(-1.0), probs)
    p2 = jnp.max(probs2, axis=1, keepdims=True)
    i2 = jnp.min(jnp.where(probs2 == p2, col, E), axis=1, keepdims=True)
    tp_ref[...] = jnp.where(col == 0, p1, jnp.where(col == 1, p2, 0.0))

    cnt = ((col == i1) | (col == i2)).astype(jnp.int32)
    csum = cnt
    sh = 1
    while sh < T:
        top = jax.lax.slice(csum, (0, 0), (T - sh, 128))
        z = jnp.zeros((sh, 128), jnp.int32)
        csum = csum + jnp.concatenate([z, top], axis=0)
        sh *= 2
    excl = csum - cnt
    counts = jax.lax.slice(csum, (T - 1, 0), (T, 128))
    nb = (counts + (BK - 1)) // BK
    lcol = lax.broadcasted_iota(jnp.int32, (128, 128), 0)
    mcol = lax.broadcasted_iota(jnp.int32, (128, 128), 1)
    tri = ((lcol <= mcol) & (lcol < E)).astype(jnp.float32)
    cnbf = jnp.dot(nb.astype(jnp.float32), tri,
                   preferred_element_type=jnp.float32)
    cnb = cnbf.astype(jnp.int32)
    pad_off = (cnb - nb) * BK
    tgt = pad_off + excl
    pe = jnp.sum(jnp.where(col == i1, tgt, 0), axis=1)
    po = jnp.sum(jnp.where(col == i2, tgt, 0), axis=1)
    pp_ref[pl.ds(0, T)] = pe
    pp_ref[pl.ds(T, T)] = po

    nused = jnp.max(jnp.where(mcol >= E - 1, cnb[0][None, :], 0))
    rowb = jnp.minimum(lcol, nused - 1)
    bemat = ((mcol < E) & (cnb[0][None, :] <= rowb)).astype(jnp.int32)
    bevec = jnp.sum(bemat, axis=1)
    bevec = jnp.where(lax.iota(jnp.int32, 128) == NB, nused, bevec)
    be_ref[...] = jax.lax.slice(bevec, (0,), (NB + 1,))


def _router(xf, wr_pad, NB):
    T = xf.shape[0]
    return pl.pallas_call(
        functools.partial(_router_body, NB=NB),
        out_shape=[jax.ShapeDtypeStruct((T, 128), jnp.float32),
                   jax.ShapeDtypeStruct((K * T,), jnp.int32),
                   jax.ShapeDtypeStruct((NB + 1,), jnp.int32)],
    )(xf, wr_pad)


def _make_dispatch(T, P, D):
    NW = 32
    tpw = T // NW
    mesh = plsc.VectorSubcoreMesh(core_axis_name="c", subcore_axis_name="s")

    @functools.partial(
        pl.kernel, mesh=mesh,
        out_type=jax.ShapeDtypeStruct((P, D), jnp.float32),
        scratch_types=[
            pltpu.VMEM((tpw,), jnp.int32),
            pltpu.VMEM((tpw,), jnp.int32),
            pltpu.VMEM((tpw, D), jnp.float32),
            pltpu.SemaphoreType.DMA,
        ],
    )
    def dk(x_hbm, pp_hbm, out_hbm, pe_v, po_v, rows_v, sem):
        wid = lax.axis_index("s") * 2 + lax.axis_index("c")
        base = wid * tpw
        pltpu.sync_copy(pp_hbm.at[pl.ds(base, tpw)], pe_v)
        pltpu.sync_copy(pp_hbm.at[pl.ds(T + base, tpw)], po_v)
        pltpu.sync_copy(x_hbm.at[pl.ds(base, tpw)], rows_v)
        c1 = pltpu.async_copy(rows_v, out_hbm.at[pe_v], sem)
        c2 = pltpu.async_copy(rows_v, out_hbm.at[po_v], sem)
        c1.wait()
        c2.wait()

    return dk


def _make_gather(N, D, C):
    NW = 32
    rows_per = N // NW
    nch = rows_per // C
    assert rows_per % C == 0 and N % NW == 0 and C % 8 == 0
    mesh = plsc.VectorSubcoreMesh(core_axis_name="c", subcore_axis_name="s")

    @functools.partial(
        pl.kernel, mesh=mesh,
        out_type=jax.ShapeDtypeStruct((N, D), jnp.float32),
        scratch_types=[
            pltpu.VMEM((C,), jnp.int32),
            pltpu.VMEM((C, D), jnp.float32),
            pltpu.SemaphoreType.DMA,
        ],
    )
    def gk(table_hbm, idx_hbm, out_hbm, idx_v, rows_v, sem):
        wid = lax.axis_index("s") * 2 + lax.axis_index("c")
        base0 = wid * rows_per

        def body(c, carry):
            base = base0 + c * C
            pltpu.sync_copy(idx_hbm.at[pl.ds(base, C)], idx_v)
            pltpu.async_copy(table_hbm.at[idx_v], rows_v, sem).wait()
            pltpu.sync_copy(rows_v, out_hbm.at[pl.ds(base, C)])
            return carry

        lax.fori_loop(0, nch, body, 0)

    return gk


def _mlp_body(be_ref, xg_ref, win_ref, bin_ref, wout_ref, bout_ref,
              out_ref, *, NB):
    @pl.when(pl.program_id(0) < be_ref[NB])
    def _():
        a = jnp.dot(xg_ref[...], win_ref[0], preferred_element_type=jnp.float32)
        a = a + bin_ref[0]
        h = jax.nn.gelu(a)
        out_ref[...] = (jnp.dot(h, wout_ref[0],
                                preferred_element_type=jnp.float32)
                        + bout_ref[0])


def _grouped_mlp(be, gathered, w_in, b_in, w_out, b_out, NB):
    P, D = gathered.shape
    H = w_in.shape[2]
    grid_spec = pltpu.PrefetchScalarGridSpec(
        num_scalar_prefetch=1,
        grid=(NB,),
        in_specs=[
            pl.BlockSpec((BK, D), lambda b, be: (jnp.minimum(b, be[NB] - 1), 0)),
            pl.BlockSpec((1, D, H), lambda b, be: (be[b], 0, 0)),
            pl.BlockSpec((1, 1, H), lambda b, be: (be[b], 0, 0)),
            pl.BlockSpec((1, H, D), lambda b, be: (be[b], 0, 0)),
            pl.BlockSpec((1, 1, D), lambda b, be: (be[b], 0, 0)),
        ],
        out_specs=pl.BlockSpec((BK, D), lambda b, be: (b, 0)),
    )
    return pl.pallas_call(
        functools.partial(_mlp_body, NB=NB),
        grid_spec=grid_spec,
        out_shape=jax.ShapeDtypeStruct((P, D), jnp.float32),
    )(be, gathered, w_in, b_in.reshape(E, 1, H), w_out,
      b_out.reshape(E, 1, D))


def _sum_body(a_ref, b_ref, g_ref, o_ref):
    g = g_ref[...]
    o_ref[...] = (a_ref[...] * jax.lax.slice(g, (0, 0), (g.shape[0], 1))
                  + b_ref[...] * jax.lax.slice(g, (0, 1), (g.shape[0], 2)))


def _pair_sum(ffu, tp, T, D):
    bt = 512
    nblk = T // bt
    return pl.pallas_call(
        _sum_body,
        grid=(nblk,),
        in_specs=[
            pl.BlockSpec((bt, D), lambda i: (i, 0)),
            pl.BlockSpec((bt, D), lambda i: (i + nblk, 0)),
            pl.BlockSpec((bt, 128), lambda i: (i, 0)),
        ],
        out_specs=pl.BlockSpec((bt, D), lambda i: (i, 0)),
        out_shape=jax.ShapeDtypeStruct((T, D), jnp.float32),
    )(ffu, ffu, tp)


def kernel(x, Wr, W_in, b_in, W_out, b_out):
    B, S, D = x.shape
    T = B * S
    TK = T * K
    NB = TK // BK + E - 1
    P = NB * BK
    xf = x.reshape(T, D)

    tp_pad, posperm, be = _router(xf, Wr, NB)

    gathered = _make_dispatch(T, P, D)(xf, posperm)

    ff = _grouped_mlp(be, gathered, W_in, b_in, W_out, b_out, NB)

    ffu = _make_gather(TK, D, 64)(ff, posperm)
    out = _pair_sum(ffu, tp_pad, T, D)
    return out.reshape(B, S, D)
